# baseline (speedup 1.0000x reference)
import functools

import jax
import jax.numpy as jnp
from jax import lax
from jax.experimental import pallas as pl
from jax.experimental.pallas import tpu as pltpu
from jax.experimental.pallas import tpu_sc as plsc

TILE = 256
META_R = 32
META_C = 128


def _meta_kernel(E, NT, te_ref, ppos_ref, tile_e_ref, act_ref):
    em = te_ref[...]
    R, C = em.shape
    ci = lax.broadcasted_iota(jnp.int32, (C, C), 0)
    cj = lax.broadcasted_iota(jnp.int32, (C, C), 1)
    U = (ci <= cj).astype(jnp.float32)
    ri = lax.broadcasted_iota(jnp.int32, (R, R), 0)
    rj = lax.broadcasted_iota(jnp.int32, (R, R), 1)
    Ls = (rj < ri).astype(jnp.float32)

    tt = lax.broadcasted_iota(jnp.int32, (1, NT), 1) * TILE
    ppos = jnp.zeros((R, C), jnp.float32)
    tile_e = jnp.zeros((1, NT), jnp.int32)
    act = jnp.zeros((1, NT), jnp.int32)
    off = jnp.int32(0)
    for e in range(E):
        m = (em == e).astype(jnp.float32)
        incl = jnp.dot(m, U, preferred_element_type=jnp.float32)
        rowsum = incl[:, C - 1 : C]
        rowpref = jnp.dot(Ls, rowsum, preferred_element_type=jnp.float32)
        rank = incl - m + rowpref
        cnt = jnp.sum(m).astype(jnp.int32)
        pad_cnt = ((cnt + TILE - 1) // TILE) * TILE
        ppos = ppos + m * (rank + off.astype(jnp.float32))
        tile_e = tile_e + (tt >= off + pad_cnt).astype(jnp.int32)
        act = act + jnp.logical_and(tt >= off, tt < off + cnt).astype(jnp.int32)
        off = off + pad_cnt
    ppos_ref[...] = ppos.astype(jnp.int32)
    tile_e_ref[...] = jnp.minimum(tile_e, E - 1)
    act_ref[...] = act


def _gmm_kernel(te_ref, act_ref, x_ref, wg_ref, wu_ref, wd_ref, y_ref):
    i = pl.program_id(0)

    @pl.when(act_ref[0, i] == 1)
    def _():
        x = x_ref[...]
        gate = jax.nn.silu(jnp.dot(x, wg_ref[0], preferred_element_type=jnp.float32))
        up = jnp.dot(x, wu_ref[0], preferred_element_type=jnp.float32)
        y_ref[...] = jnp.dot(gate * up, wd_ref[0], preferred_element_type=jnp.float32)


def _dispatch_body(T, NC, x_hbm, ppos_hbm, xs_hbm, rows_v, idx0_v, idx1_v,
                   sem0, sem1):
    wid = lax.axis_index("s") * NC + lax.axis_index("c")
    tb = wid * 64
    pltpu.sync_copy(ppos_hbm.at[pl.ds(tb, 64)], idx0_v)
    pltpu.sync_copy(ppos_hbm.at[pl.ds(T + tb, 64)], idx1_v)
    pltpu.sync_copy(x_hbm.at[pl.ds(tb, 64)], rows_v)
    c0 = pltpu.async_copy(rows_v, xs_hbm.at[idx0_v], sem0)
    c1 = pltpu.async_copy(rows_v, xs_hbm.at[idx1_v], sem1)
    c0.wait()
    c1.wait()


def _combine_body(T, NC, H, ppos_hbm, wb_hbm, ys_hbm, out_hbm,
                  idx0_v, idx1_v, w0_v, w1_v, a_v, b_v, o_v,
                  semA0, semA1, semB0, semB1):
    wid = lax.axis_index("s") * NC + lax.axis_index("c")
    NV = H // 16
    CH = 16
    tb = wid * 64
    pltpu.sync_copy(ppos_hbm.at[pl.ds(tb, 64)], idx0_v)
    pltpu.sync_copy(ppos_hbm.at[pl.ds(T + tb, 64)], idx1_v)
    pltpu.sync_copy(wb_hbm.at[pl.ds(tb, 64)], w0_v)
    pltpu.sync_copy(wb_hbm.at[pl.ds(T + tb, 64)], w1_v)
    semA = (semA0, semA1)
    semB = (semB0, semB1)

    def fire(ch):
        s = ch % 2
        ca = pltpu.async_copy(
            ys_hbm.at[idx0_v.at[pl.ds(ch * CH, CH)]], a_v.at[s], semA[s]
        )
        cb = pltpu.async_copy(
            ys_hbm.at[idx1_v.at[pl.ds(ch * CH, CH)]], b_v.at[s], semB[s]
        )
        return ca, cb

    pend = fire(0)
    for ch in range(4):
        nxt = fire(ch + 1) if ch + 1 < 4 else None
        pend[0].wait()
        pend[1].wait()
        s = ch % 2

        def body(i, carry):
            w0 = w0_v[ch * CH + i, :]
            w1 = w1_v[ch * CH + i, :]
            for v in range(NV):
                sl = pl.ds(v * 16, 16)
                o_v[i, sl] = a_v[s, i, sl] * w0 + b_v[s, i, sl] * w1
            return carry

        lax.fori_loop(0, CH, body, 0)
        pltpu.sync_copy(o_v, out_hbm.at[pl.ds(tb + ch * CH, CH)])
        pend = nxt


def kernel(hidden_states, top_weights, top_experts, Wg, Wu, Wd):
    B, S, H = hidden_states.shape
    T = B * S
    E, _, F = Wg.shape
    K = top_weights.shape[1]
    TK = T * K
    NT = TK // TILE + E
    P = NT * TILE
    x = hidden_states.reshape(T, H)

    te_t = top_experts.astype(jnp.int32).T.reshape(META_R, META_C)
    w_b = jnp.broadcast_to(top_weights.T.reshape(TK, 1), (TK, 16))

    ppos, tile_e, act = pl.pallas_call(
        functools.partial(_meta_kernel, E, NT),
        out_shape=(
            jax.ShapeDtypeStruct((META_R, META_C), jnp.int32),
            jax.ShapeDtypeStruct((1, NT), jnp.int32),
            jax.ShapeDtypeStruct((1, NT), jnp.int32),
        ),
    )(te_t)
    ppos_flat = ppos.reshape(TK)

    info = plsc.get_sparse_core_info()
    NC = info.num_cores
    mesh = plsc.VectorSubcoreMesh(core_axis_name="c", subcore_axis_name="s")

    x_sorted = pl.kernel(
        functools.partial(_dispatch_body, T, NC),
        out_type=jax.ShapeDtypeStruct((P, H), jnp.float32),
        mesh=mesh,
        scratch_types=[
            pltpu.VMEM((64, H), jnp.float32),
            pltpu.VMEM((64,), jnp.int32),
            pltpu.VMEM((64,), jnp.int32),
            pltpu.SemaphoreType.DMA,
            pltpu.SemaphoreType.DMA,
        ],
    )(x, ppos_flat)

    grid_spec = pltpu.PrefetchScalarGridSpec(
        num_scalar_prefetch=2,
        grid=(NT,),
        in_specs=[
            pl.BlockSpec((TILE, H), lambda i, te_m, act_m: (i, 0)),
            pl.BlockSpec((1, H, F), lambda i, te_m, act_m: (te_m[0, i], 0, 0)),
            pl.BlockSpec((1, H, F), lambda i, te_m, act_m: (te_m[0, i], 0, 0)),
            pl.BlockSpec((1, F, H), lambda i, te_m, act_m: (te_m[0, i], 0, 0)),
        ],
        out_specs=pl.BlockSpec((TILE, H), lambda i, te_m, act_m: (i, 0)),
    )
    y_s = pl.pallas_call(
        _gmm_kernel,
        grid_spec=grid_spec,
        out_shape=jax.ShapeDtypeStruct((P, H), jnp.float32),
    )(tile_e, act, x_sorted, Wg, Wu, Wd)

    out = pl.kernel(
        functools.partial(_combine_body, T, NC, H),
        out_type=jax.ShapeDtypeStruct((T, H), jnp.float32),
        mesh=mesh,
        scratch_types=[
            pltpu.VMEM((64,), jnp.int32),
            pltpu.VMEM((64,), jnp.int32),
            pltpu.VMEM((64, 16), jnp.float32),
            pltpu.VMEM((64, 16), jnp.float32),
            pltpu.VMEM((2, 16, H), jnp.float32),
            pltpu.VMEM((2, 16, H), jnp.float32),
            pltpu.VMEM((16, H), jnp.float32),
            pltpu.SemaphoreType.DMA,
            pltpu.SemaphoreType.DMA,
            pltpu.SemaphoreType.DMA,
            pltpu.SemaphoreType.DMA,
        ],
    )(ppos_flat, w_b, y_s)

    return out.reshape(B, S, H)
